# baseline (speedup 1.0000x reference)
import functools
import jax
import jax.numpy as jnp
from jax import lax
from jax.experimental import pallas as pl
from jax.experimental.pallas import tpu as pltpu
from jax.experimental.pallas import tpu_sc as plsc

RESO = 128
L = 16
NC = 2
NS = 16
NW = NC * NS
CHUNK = 4096
GROUPS = CHUNK // L
B1 = 131072


def _tc_pad_batch(arrays, P_pad):
    P = arrays[0].shape[0]
    nin = -(-P // B1)
    n = len(arrays)

    def body(*refs):
        ins, outs = refs[:n], refs[n:]
        for i_ref, o_ref in zip(ins, outs):
            o_ref[...] = i_ref[...]

    return pl.pallas_call(
        body,
        grid=(P_pad // B1,),
        in_specs=[pl.BlockSpec((B1,), lambda i: (jnp.minimum(i, nin - 1),))
                  for _ in arrays],
        out_specs=[pl.BlockSpec((B1,), lambda i: (i,)) for _ in arrays],
        out_shape=[jax.ShapeDtypeStruct((P_pad,), a.dtype) for a in arrays],
    )(*arrays)


def _tc_slice_batch(arrays, P):
    n = len(arrays)

    def body(*refs):
        ins, outs = refs[:n], refs[n:]
        for i_ref, o_ref in zip(ins, outs):
            o_ref[...] = i_ref[...]

    return pl.pallas_call(
        body,
        grid=(-(-P // B1),),
        in_specs=[pl.BlockSpec((B1,), lambda i: (i,)) for _ in arrays],
        out_specs=[pl.BlockSpec((B1,), lambda i: (i,)) for _ in arrays],
        out_shape=[jax.ShapeDtypeStruct((P,), a.dtype) for a in arrays],
    )(*arrays)


def _quadric_list_sc(P_pad, want_normal):
    s_max = P_pad // (NW * CHUNK)
    mesh = plsc.VectorSubcoreMesh(core_axis_name="c", subcore_axis_name="s",
                                  num_cores=NC, num_subcores=NS)
    out1 = jax.ShapeDtypeStruct((P_pad,), jnp.float32)
    buf_f = pltpu.VMEM((CHUNK,), jnp.float32)
    buf_i = pltpu.VMEM((CHUNK,), jnp.int32)

    @functools.partial(
        pl.kernel,
        out_type=(out1, out1, out1) if want_normal else (out1,),
        mesh=mesh,
        compiler_params=pltpu.CompilerParams(needs_layout_passes=False),
        scratch_types=dict(
            xl=pltpu.VMEM((RESO,), jnp.float32),
            yl=pltpu.VMEM((RESO,), jnp.float32),
            zl=pltpu.VMEM((RESO,), jnp.float32),
            off=pltpu.VMEM((RESO,), jnp.float32),
            idx_v=[buf_i, buf_i],
            px=[buf_f, buf_f],
            py=[buf_f, buf_f],
            pz=[buf_f, buf_f],
            ox=[buf_f, buf_f],
            oy=[buf_f, buf_f],
            oz=[buf_f, buf_f],
            in_sem=[pltpu.SemaphoreType.DMA, pltpu.SemaphoreType.DMA],
            out_sem=[pltpu.SemaphoreType.DMA, pltpu.SemaphoreType.DMA],
        ),
    )
    def k(*refs, xl, yl, zl, off, idx_v, px, py, pz, ox, oy, oz,
          in_sem, out_sem):
        (x_hbm, y_hbm, z_hbm, idx_hbm, xl_hbm, yl_hbm, zl_hbm, off_hbm) = refs[:8]
        outs = refs[8:]
        wid = lax.axis_index("s") * NC + lax.axis_index("c")

        pltpu.sync_copy(xl_hbm, xl)
        pltpu.sync_copy(yl_hbm, yl)
        pltpu.sync_copy(zl_hbm, zl)
        pltpu.sync_copy(off_hbm, off)

        d = off[pl.ds(0 * 2 * L, L)]
        e = off[pl.ds(1 * 2 * L, L)]
        f = off[pl.ds(2 * 2 * L, L)]
        g = off[pl.ds(3 * 2 * L, L)]

        def issue_loads(u, b):
            sl = pl.ds((wid * s_max + u) * CHUNK, CHUNK)
            return [pltpu.async_copy(s.at[sl], t, in_sem[b])
                    for s, t in zip((idx_hbm, x_hbm, y_hbm, z_hbm),
                                    (idx_v[b], px[b], py[b], pz[b]))]

        def compute(b):
            @pl.loop(0, GROUPS)
            def _(grp):
                sl = pl.ds(grp * L, L)
                idx = idx_v[b][sl]
                ii = lax.shift_right_logical(idx, 14) & (RESO - 1)
                jj = lax.shift_right_logical(idx, 7) & (RESO - 1)
                kk = idx & (RESO - 1)
                a = plsc.load_gather(xl, [ii])
                bb = plsc.load_gather(yl, [jj])
                c = plsc.load_gather(zl, [kk])
                if want_normal:
                    ox[b][sl] = (a + a) * px[b][sl] + d
                    oy[b][sl] = (bb + bb) * py[b][sl] + e
                    oz[b][sl] = (c + c) * pz[b][sl] + f
                else:
                    x = px[b][sl]
                    y = py[b][sl]
                    z = pz[b][sl]
                    ox[b][sl] = (a * x * x + bb * y * y + c * z * z
                                 + d * x + e * y + f * z + g)

        def issue_stores(u, b):
            sl = pl.ds((wid * s_max + u) * CHUNK, CHUNK)
            srcs = (ox[b], oy[b], oz[b]) if want_normal else (ox[b],)
            return [pltpu.async_copy(s, o.at[sl], out_sem[b])
                    for s, o in zip(srcs, outs)]

        loads = {0: issue_loads(0, 0)}
        stores = {}
        for u in range(s_max):
            b = u % 2
            if u + 1 < s_max:
                loads[u + 1] = issue_loads(u + 1, (u + 1) % 2)
            for dsc in loads.pop(u):
                dsc.wait()
            if u - 2 >= 0:
                for dsc in stores.pop(u - 2):
                    dsc.wait()
            compute(b)
            stores[u] = issue_stores(u, b)
        for u in (s_max - 2, s_max - 1):
            for dsc in stores.pop(u):
                dsc.wait()

    return k


@jax.jit
def kernel(renderPointList, renderIndexList, sdfPointList, sdfIndexList,
           xLayer, yLayer, zLayer, offset):
    P = renderPointList.shape[0]
    work = NW * CHUNK
    P_pad = -(-P // work) * work
    k_sdf = _quadric_list_sc(P_pad, want_normal=False)
    k_nrm = _quadric_list_sc(P_pad, want_normal=True)
    off128 = jnp.repeat(offset, 2 * L)
    rx, ry, rz, ridx = _tc_pad_batch(
        [renderPointList[:, 0], renderPointList[:, 1], renderPointList[:, 2],
         renderIndexList], P_pad)
    nx, ny, nz = k_nrm(rx, ry, rz, ridx, xLayer, yLayer, zLayer, off128)
    sx, sy, sz, sidx = _tc_pad_batch(
        [sdfPointList[:, 0], sdfPointList[:, 1], sdfPointList[:, 2],
         sdfIndexList], P_pad)
    (sdf,) = k_sdf(sx, sy, sz, sidx, xLayer, yLayer, zLayer, off128)
    nx_o, ny_o, nz_o = _tc_slice_batch([nx, ny, nz], P)
    nrm = jnp.stack([nx_o, ny_o, nz_o], axis=1)
    (sdf_o,) = _tc_slice_batch([sdf], P)
    return sdf_o, nrm


# speedup vs baseline: 1.7035x; 1.7035x over previous
import functools
import jax
import jax.numpy as jnp
from jax import lax
from jax.experimental import pallas as pl
from jax.experimental.pallas import tpu as pltpu
from jax.experimental.pallas import tpu_sc as plsc

RESO = 128
L = 16
NC = 2
NS = 16
NW = NC * NS
CHUNK = 4096
GROUPS = CHUNK // L
B1 = 131072


def _tc_pad_batch(arrays, P_pad):
    P = arrays[0].shape[0]
    nin = -(-P // B1)
    n = len(arrays)

    def body(*refs):
        ins, outs = refs[:n], refs[n:]
        for i_ref, o_ref in zip(ins, outs):
            o_ref[...] = i_ref[...]

    return pl.pallas_call(
        body,
        grid=(P_pad // B1,),
        in_specs=[pl.BlockSpec((B1,), lambda i: (jnp.minimum(i, nin - 1),))
                  for _ in arrays],
        out_specs=[pl.BlockSpec((B1,), lambda i: (i,)) for _ in arrays],
        out_shape=[jax.ShapeDtypeStruct((P_pad,), a.dtype) for a in arrays],
    )(*arrays)


def _tc_slice_batch(arrays, P):
    n = len(arrays)

    def body(*refs):
        ins, outs = refs[:n], refs[n:]
        for i_ref, o_ref in zip(ins, outs):
            o_ref[...] = i_ref[...]

    return pl.pallas_call(
        body,
        grid=(-(-P // B1),),
        in_specs=[pl.BlockSpec((B1,), lambda i: (i,)) for _ in arrays],
        out_specs=[pl.BlockSpec((B1,), lambda i: (i,)) for _ in arrays],
        out_shape=[jax.ShapeDtypeStruct((P,), a.dtype) for a in arrays],
    )(*arrays)


def _quadric_list_sc(P_pad, want_normal):
    s_max = P_pad // (NW * CHUNK)
    mesh = plsc.VectorSubcoreMesh(core_axis_name="c", subcore_axis_name="s",
                                  num_cores=NC, num_subcores=NS)
    out1 = (jax.ShapeDtypeStruct((4 * P_pad,), jnp.float32) if want_normal
            else jax.ShapeDtypeStruct((P_pad,), jnp.float32))
    buf_f = pltpu.VMEM((4 * CHUNK,), jnp.float32)
    buf_i = pltpu.VMEM((CHUNK,), jnp.int32)

    @functools.partial(
        pl.kernel,
        out_type=(out1,),
        mesh=mesh,
        compiler_params=pltpu.CompilerParams(needs_layout_passes=False),
        scratch_types=dict(
            xl=pltpu.VMEM((RESO,), jnp.float32),
            yl=pltpu.VMEM((RESO,), jnp.float32),
            zl=pltpu.VMEM((RESO,), jnp.float32),
            off=pltpu.VMEM((RESO,), jnp.float32),
            idx_v=[buf_i, buf_i],
            px=[buf_f, buf_f],
            ox=[buf_f, buf_f],
            in_sem=[pltpu.SemaphoreType.DMA, pltpu.SemaphoreType.DMA],
            out_sem=[pltpu.SemaphoreType.DMA, pltpu.SemaphoreType.DMA],
        ),
    )
    def k(*refs, xl, yl, zl, off, idx_v, px, ox, in_sem, out_sem):
        (pts_hbm, idx_hbm, xl_hbm, yl_hbm, zl_hbm, off_hbm) = refs[:6]
        outs = refs[6:]
        wid = lax.axis_index("s") * NC + lax.axis_index("c")

        pltpu.sync_copy(xl_hbm, xl)
        pltpu.sync_copy(yl_hbm, yl)
        pltpu.sync_copy(zl_hbm, zl)
        pltpu.sync_copy(off_hbm, off)

        d = off[pl.ds(0 * 2 * L, L)]
        e = off[pl.ds(1 * 2 * L, L)]
        f = off[pl.ds(2 * 2 * L, L)]
        g = off[pl.ds(3 * 2 * L, L)]

        def issue_loads(u, b):
            base = (wid * s_max + u) * CHUNK
            return [pltpu.async_copy(idx_hbm.at[pl.ds(base, CHUNK)],
                                     idx_v[b], in_sem[b]),
                    pltpu.async_copy(pts_hbm.at[pl.ds(4 * base, 4 * CHUNK)],
                                     px[b], in_sem[b])]

        def compute(b):
            @pl.loop(0, GROUPS)
            def _(grp):
                sl = pl.ds(grp * L, L)
                toff = lax.shift_left(lax.shift_right_logical(grp, 3), 9) \
                    + lax.shift_left(grp & 7, 4)
                idx = idx_v[b][sl]
                ii = lax.shift_right_logical(idx, 14) & (RESO - 1)
                jj = lax.shift_right_logical(idx, 7) & (RESO - 1)
                kk = idx & (RESO - 1)
                a = plsc.load_gather(xl, [ii])
                bb = plsc.load_gather(yl, [jj])
                c = plsc.load_gather(zl, [kk])
                x = px[b][pl.ds(toff, L)]
                y = px[b][pl.ds(toff + 128, L)]
                z = px[b][pl.ds(toff + 256, L)]
                if want_normal:
                    ox[b][pl.ds(toff, L)] = (a + a) * x + d
                    ox[b][pl.ds(toff + 128, L)] = (bb + bb) * y + e
                    ox[b][pl.ds(toff + 256, L)] = (c + c) * z + f
                else:
                    ox[b][sl] = (a * x * x + bb * y * y + c * z * z
                                 + d * x + e * y + f * z + g)

        def issue_stores(u, b):
            base = (wid * s_max + u) * CHUNK
            if want_normal:
                return [pltpu.async_copy(
                    ox[b], outs[0].at[pl.ds(4 * base, 4 * CHUNK)], out_sem[b])]
            return [pltpu.async_copy(
                ox[b].at[pl.ds(0, CHUNK)], outs[0].at[pl.ds(base, CHUNK)],
                out_sem[b])]

        loads = {0: issue_loads(0, 0)}
        stores = {}
        for u in range(s_max):
            b = u % 2
            if u + 1 < s_max:
                loads[u + 1] = issue_loads(u + 1, (u + 1) % 2)
            for dsc in loads.pop(u):
                dsc.wait()
            if u - 2 >= 0:
                for dsc in stores.pop(u - 2):
                    dsc.wait()
            compute(b)
            stores[u] = issue_stores(u, b)
        for u in (s_max - 2, s_max - 1):
            for dsc in stores.pop(u):
                dsc.wait()

    return k


@jax.jit
def kernel(renderPointList, renderIndexList, sdfPointList, sdfIndexList,
           xLayer, yLayer, zLayer, offset):
    P = renderPointList.shape[0]
    work = NW * CHUNK
    P_pad = -(-P // work) * work
    k_sdf = _quadric_list_sc(P_pad, want_normal=False)
    k_nrm = _quadric_list_sc(P_pad, want_normal=True)
    off128 = jnp.repeat(offset, 2 * L)
    def to_tiles(pts):
        q = jnp.pad(pts, ((0, P_pad - P), (0, 1)))
        return q.reshape(P_pad // 128, 128, 4).transpose(0, 2, 1).reshape(-1)

    rp = to_tiles(renderPointList)
    (ridx,) = _tc_pad_batch([renderIndexList], P_pad)
    (nrm4,) = k_nrm(rp, ridx, xLayer, yLayer, zLayer, off128)
    sp = to_tiles(sdfPointList)
    (sidx,) = _tc_pad_batch([sdfIndexList], P_pad)
    (sdf,) = k_sdf(sp, sidx, xLayer, yLayer, zLayer, off128)
    nrm = (nrm4.reshape(P_pad // 128, 4, 128).transpose(0, 2, 1)
           .reshape(P_pad, 4)[:P, :3])
    (sdf_o,) = _tc_slice_batch([sdf], P)
    return sdf_o, nrm
